# baseline (speedup 1.0000x reference)
import functools

import jax
import jax.numpy as jnp
import numpy as np
from jax.experimental import pallas as pl
from jax.experimental.pallas import tpu as pltpu

B = 2
N = 4096
D_IN = 128
NPOINT = 512
NSAMPLE = 32
D_OUT = 256
NHEAD = 8
NLAYERS = 4
EPS = 1e-5


def _fps(xyz, npoint):
    Bb, Nn, _ = xyz.shape
    bidx = jnp.arange(Bb)

    def body(i, state):
        centroids, distance, farthest = state
        centroids = centroids.at[:, i].set(farthest)
        centroid = xyz[bidx, farthest][:, None, :]
        dist = jnp.sum((xyz - centroid) ** 2, axis=-1)
        distance = jnp.minimum(distance, dist)
        farthest = jnp.argmax(distance, axis=-1).astype(jnp.int32)
        return (centroids, distance, farthest)

    state = (jnp.zeros((Bb, npoint), jnp.int32),
             jnp.full((Bb, Nn), 1e10, jnp.float32),
             jnp.zeros((Bb,), jnp.int32))
    centroids, _, _ = jax.lax.fori_loop(0, npoint, body, state)
    return centroids


def _index_points(points, idx):
    b = jnp.arange(points.shape[0]).reshape((-1,) + (1,) * (idx.ndim - 1))
    return points[b, idx]


def _knn_point(nsample, xyz, new_xyz):
    d = (jnp.sum(new_xyz ** 2, -1)[:, :, None]
         + jnp.sum(xyz ** 2, -1)[:, None, :]
         - 2.0 * jnp.einsum('bsc,bnc->bsn', new_xyz, xyz))
    _, idx = jax.lax.top_k(-d, nsample)
    return idx


def _layer_norm(x, g, b):
    m = jnp.mean(x, -1, keepdims=True)
    v = jnp.var(x, -1, keepdims=True)
    return (x - m) / jnp.sqrt(v + EPS) * g + b


def _mha(x, p):
    S, T, D = x.shape
    hd = D // NHEAD
    qkv = jnp.einsum('std,ed->ste', x, p['w_qkv']) + p['b_qkv']
    q, k, v = jnp.split(qkv, 3, axis=-1)

    def heads(a):
        return a.reshape(S, T, NHEAD, hd).transpose(1, 2, 0, 3)

    q, k, v = heads(q), heads(k), heads(v)
    att = jax.nn.softmax(jnp.einsum('thsd,thud->thsu', q, k) / jnp.sqrt(float(hd)), axis=-1)
    o = jnp.einsum('thsu,thud->thsd', att, v).transpose(2, 0, 1, 3).reshape(S, T, D)
    return jnp.einsum('std,ed->ste', o, p['w_o']) + p['b_o']


def _encoder_layer(x, p):
    x = _layer_norm(x + _mha(x, p), p['ln1_g'], p['ln1_b'])
    h = jax.nn.relu(jnp.einsum('std,ed->ste', x, p['w_ff1']) + p['b_ff1'])
    ff = jnp.einsum('ste,de->std', h, p['w_ff2']) + p['b_ff2']
    return _layer_norm(x + ff, p['ln2_g'], p['ln2_b'])


def _conv1x1(x, w, b):
    return jnp.einsum('oc,bc...->bo...', w, x) + b.reshape((1, -1) + (1,) * (x.ndim - 2))


def _pe_net(gx, params):
    h = _conv1x1(gx, params['pe_w1'], params['pe_b1'])
    h = h / jnp.sqrt(1.0 + EPS) * params['pe_bn_g'].reshape(1, -1, 1, 1) + params['pe_bn_b'].reshape(1, -1, 1, 1)
    h = jax.nn.relu(h)
    return _conv1x1(h, params['pe_w2'], params['pe_b2'])


def _lin_interp(x, out_size):
    L = x.shape[-1]
    pos = jnp.arange(out_size) * ((L - 1) / (out_size - 1))
    lo = jnp.floor(pos).astype(jnp.int32)
    hi = jnp.minimum(lo + 1, L - 1)
    w = (pos - lo).astype(x.dtype)
    return x[..., lo] * (1.0 - w) + x[..., hi] * w


def _identity_kernel(x_ref, o_ref):
    o_ref[...] = x_ref[...]


def _pallas_identity(x):
    return pl.pallas_call(
        _identity_kernel,
        out_shape=jax.ShapeDtypeStruct(x.shape, x.dtype),
    )(x)


def kernel(xyz, features, params):
    xyzf = xyz.transpose(0, 2, 1)
    fps_idx = _fps(xyzf, NPOINT)
    new_xyz = _index_points(xyzf, fps_idx)
    group_idx = _knn_point(NSAMPLE, xyzf, new_xyz)

    grouped_xyz = _index_points(xyzf, group_idx).transpose(0, 3, 1, 2)
    grouped_feat = _index_points(features.transpose(0, 2, 1), group_idx).transpose(0, 3, 1, 2)
    inp = grouped_feat + _pe_net(grouped_xyz, params)
    Bb, D, S, K = inp.shape
    x = inp.transpose(0, 2, 1, 3).reshape(Bb * S, D, K).transpose(2, 0, 1)
    for p in params['layers']:
        x = _encoder_layer(x, p)
    tf = x.transpose(1, 2, 0).reshape(Bb, S, D, K).transpose(0, 2, 1, 3)
    pooled = jnp.max(tf, axis=-1)
    out = _conv1x1(pooled, params['fc_w'], params['fc_b'])
    up = _lin_interp(out, N)
    up = _pallas_identity(up)
    return (new_xyz.transpose(0, 2, 1), up)


# speedup vs baseline: 1.3565x; 1.3565x over previous
import functools

import jax
import jax.numpy as jnp
import numpy as np
from jax.experimental import pallas as pl
from jax.experimental.pallas import tpu as pltpu

B = 2
N = 4096
D_IN = 128
NPOINT = 512
NSAMPLE = 32
D_OUT = 256
NHEAD = 8
NLAYERS = 4
EPS = 1e-5


_FR, _FC = 32, 128
_IR, _IC = 4, 128


def _fps_kernel(x_ref, y_ref, z_ref, idx_ref):
    x = x_ref[...]
    y = y_ref[...]
    z = z_ref[...]
    flat = (jax.lax.broadcasted_iota(jnp.int32, (B, _FR, _FC), 1) * _FC
            + jax.lax.broadcasted_iota(jnp.int32, (B, _FR, _FC), 2))
    oflat = (jax.lax.broadcasted_iota(jnp.int32, (B, _IR, _IC), 1) * _IC
             + jax.lax.broadcasted_iota(jnp.int32, (B, _IR, _IC), 2))

    def body(i, state):
        dmin, f, acc = state
        acc = jnp.where(oflat == i, f[:, None, None].astype(jnp.int32), acc)
        sel = flat == f[:, None, None]
        cx = jnp.sum(jnp.where(sel, x, 0.0), axis=(1, 2))
        cy = jnp.sum(jnp.where(sel, y, 0.0), axis=(1, 2))
        cz = jnp.sum(jnp.where(sel, z, 0.0), axis=(1, 2))
        dx = x - cx[:, None, None]
        dy = y - cy[:, None, None]
        dz = z - cz[:, None, None]
        dist = dx * dx + dy * dy + dz * dz
        dmin = jnp.minimum(dmin, dist)
        m = jnp.max(dmin, axis=(1, 2))
        fnew = jnp.min(jnp.where(dmin == m[:, None, None], flat, N), axis=(1, 2))
        return dmin, fnew, acc

    dmin0 = jnp.full((B, _FR, _FC), 1e10, jnp.float32)
    f0 = jnp.zeros((B,), jnp.int32)
    acc0 = jnp.zeros((B, _IR, _IC), jnp.int32)
    _, _, acc = jax.lax.fori_loop(0, NPOINT, body, (dmin0, f0, acc0))
    idx_ref[...] = acc


def _fps(xyzf, npoint):
    x = xyzf[..., 0].reshape(B, _FR, _FC)
    y = xyzf[..., 1].reshape(B, _FR, _FC)
    z = xyzf[..., 2].reshape(B, _FR, _FC)
    idx = pl.pallas_call(
        _fps_kernel,
        out_shape=jax.ShapeDtypeStruct((B, _IR, _IC), jnp.int32),
    )(x, y, z)
    return idx.reshape(B, NPOINT)


def _index_points(points, idx):
    b = jnp.arange(points.shape[0]).reshape((-1,) + (1,) * (idx.ndim - 1))
    return points[b, idx]


def _knn_point(nsample, xyz, new_xyz):
    d = (jnp.sum(new_xyz ** 2, -1)[:, :, None]
         + jnp.sum(xyz ** 2, -1)[:, None, :]
         - 2.0 * jnp.einsum('bsc,bnc->bsn', new_xyz, xyz))
    _, idx = jax.lax.top_k(-d, nsample)
    return idx


def _layer_norm(x, g, b):
    m = jnp.mean(x, -1, keepdims=True)
    v = jnp.var(x, -1, keepdims=True)
    return (x - m) / jnp.sqrt(v + EPS) * g + b


def _mha(x, p):
    S, T, D = x.shape
    hd = D // NHEAD
    qkv = jnp.einsum('std,ed->ste', x, p['w_qkv']) + p['b_qkv']
    q, k, v = jnp.split(qkv, 3, axis=-1)

    def heads(a):
        return a.reshape(S, T, NHEAD, hd).transpose(1, 2, 0, 3)

    q, k, v = heads(q), heads(k), heads(v)
    att = jax.nn.softmax(jnp.einsum('thsd,thud->thsu', q, k) / jnp.sqrt(float(hd)), axis=-1)
    o = jnp.einsum('thsu,thud->thsd', att, v).transpose(2, 0, 1, 3).reshape(S, T, D)
    return jnp.einsum('std,ed->ste', o, p['w_o']) + p['b_o']


def _encoder_layer(x, p):
    x = _layer_norm(x + _mha(x, p), p['ln1_g'], p['ln1_b'])
    h = jax.nn.relu(jnp.einsum('std,ed->ste', x, p['w_ff1']) + p['b_ff1'])
    ff = jnp.einsum('ste,de->std', h, p['w_ff2']) + p['b_ff2']
    return _layer_norm(x + ff, p['ln2_g'], p['ln2_b'])


def _conv1x1(x, w, b):
    return jnp.einsum('oc,bc...->bo...', w, x) + b.reshape((1, -1) + (1,) * (x.ndim - 2))


def _pe_net(gx, params):
    h = _conv1x1(gx, params['pe_w1'], params['pe_b1'])
    h = h / jnp.sqrt(1.0 + EPS) * params['pe_bn_g'].reshape(1, -1, 1, 1) + params['pe_bn_b'].reshape(1, -1, 1, 1)
    h = jax.nn.relu(h)
    return _conv1x1(h, params['pe_w2'], params['pe_b2'])


def _lin_interp(x, out_size):
    L = x.shape[-1]
    pos = jnp.arange(out_size) * ((L - 1) / (out_size - 1))
    lo = jnp.floor(pos).astype(jnp.int32)
    hi = jnp.minimum(lo + 1, L - 1)
    w = (pos - lo).astype(x.dtype)
    return x[..., lo] * (1.0 - w) + x[..., hi] * w


def _identity_kernel(x_ref, o_ref):
    o_ref[...] = x_ref[...]


def _pallas_identity(x):
    return pl.pallas_call(
        _identity_kernel,
        out_shape=jax.ShapeDtypeStruct(x.shape, x.dtype),
    )(x)


def kernel(xyz, features, params):
    xyzf = xyz.transpose(0, 2, 1)
    fps_idx = _fps(xyzf, NPOINT)
    new_xyz = _index_points(xyzf, fps_idx)
    group_idx = _knn_point(NSAMPLE, xyzf, new_xyz)

    grouped_xyz = _index_points(xyzf, group_idx).transpose(0, 3, 1, 2)
    grouped_feat = _index_points(features.transpose(0, 2, 1), group_idx).transpose(0, 3, 1, 2)
    inp = grouped_feat + _pe_net(grouped_xyz, params)
    Bb, D, S, K = inp.shape
    x = inp.transpose(0, 2, 1, 3).reshape(Bb * S, D, K).transpose(2, 0, 1)
    for p in params['layers']:
        x = _encoder_layer(x, p)
    tf = x.transpose(1, 2, 0).reshape(Bb, S, D, K).transpose(0, 2, 1, 3)
    pooled = jnp.max(tf, axis=-1)
    out = _conv1x1(pooled, params['fc_w'], params['fc_b'])
    up = _lin_interp(out, N)
    up = _pallas_identity(up)
    return (new_xyz.transpose(0, 2, 1), up)


# speedup vs baseline: 1.6259x; 1.1985x over previous
import functools

import jax
import jax.numpy as jnp
import numpy as np
from jax.experimental import pallas as pl
from jax.experimental.pallas import tpu as pltpu

B = 2
N = 4096
D_IN = 128
NPOINT = 512
NSAMPLE = 32
D_OUT = 256
NHEAD = 8
NLAYERS = 4
EPS = 1e-5


_FR, _FC = 32, 128
_IR, _IC = 4, 128


def _fps_kernel(x_ref, y_ref, z_ref, idx_ref):
    x = x_ref[...]
    y = y_ref[...]
    z = z_ref[...]
    flat = (jax.lax.broadcasted_iota(jnp.int32, (B, _FR, _FC), 1) * _FC
            + jax.lax.broadcasted_iota(jnp.int32, (B, _FR, _FC), 2))
    oflat = (jax.lax.broadcasted_iota(jnp.int32, (B, _IR, _IC), 1) * _IC
             + jax.lax.broadcasted_iota(jnp.int32, (B, _IR, _IC), 2))

    def body(i, state):
        dmin, f, acc = state
        acc = jnp.where(oflat == i, f[:, None, None].astype(jnp.int32), acc)
        sel = flat == f[:, None, None]
        cx = jnp.sum(jnp.where(sel, x, 0.0), axis=(1, 2))
        cy = jnp.sum(jnp.where(sel, y, 0.0), axis=(1, 2))
        cz = jnp.sum(jnp.where(sel, z, 0.0), axis=(1, 2))
        dx = x - cx[:, None, None]
        dy = y - cy[:, None, None]
        dz = z - cz[:, None, None]
        dist = dx * dx + dy * dy + dz * dz
        dmin = jnp.minimum(dmin, dist)
        m = jnp.max(dmin, axis=(1, 2))
        fnew = jnp.min(jnp.where(dmin == m[:, None, None], flat, N), axis=(1, 2))
        return dmin, fnew, acc

    dmin0 = jnp.full((B, _FR, _FC), 1e10, jnp.float32)
    f0 = jnp.zeros((B,), jnp.int32)
    acc0 = jnp.zeros((B, _IR, _IC), jnp.int32)
    _, _, acc = jax.lax.fori_loop(0, NPOINT, body, (dmin0, f0, acc0))
    idx_ref[...] = acc


def _fps(xyzf, npoint):
    x = xyzf[..., 0].reshape(B, _FR, _FC)
    y = xyzf[..., 1].reshape(B, _FR, _FC)
    z = xyzf[..., 2].reshape(B, _FR, _FC)
    idx = pl.pallas_call(
        _fps_kernel,
        out_shape=jax.ShapeDtypeStruct((B, _IR, _IC), jnp.int32),
    )(x, y, z)
    return idx.reshape(B, NPOINT)


def _index_points(points, idx):
    b = jnp.arange(points.shape[0]).reshape((-1,) + (1,) * (idx.ndim - 1))
    return points[b, idx]


_FBIG = float(np.finfo(np.float32).max)


def _knn_kernel(xrow_ref, yrow_ref, zrow_ref, tab_ref, idxcol_ref, gi_ref, q_ref):
    xrow = xrow_ref[0]
    yrow = yrow_ref[0]
    zrow = zrow_ref[0]
    tab = tab_ref[0]
    idxcol = idxcol_ref[0]
    lane = jax.lax.broadcasted_iota(jnp.int32, (1, N), 1)

    oh = jnp.where(idxcol == lane, 1.0, 0.0)
    q = jnp.dot(oh, tab, preferred_element_type=jnp.float32)
    qx, qy, qz = q[:, 0:1], q[:, 1:2], q[:, 2:3]
    qn = qx * qx + qy * qy + qz * qz
    xn = xrow * xrow + yrow * yrow + zrow * zrow
    cross = qx * xrow + qy * yrow + qz * zrow
    d = (qn + xn) - 2.0 * cross

    kcol = jax.lax.broadcasted_iota(jnp.int32, (NPOINT, NSAMPLE), 1)

    def body(k, state):
        v, i, acc = state
        taken = (d < v) | ((d == v) & (lane <= i))
        mod = jnp.where(taken, _FBIG, d)
        m = jnp.min(mod, axis=-1, keepdims=True)
        sel = jnp.min(jnp.where(mod == m, lane, N), axis=-1, keepdims=True)
        acc = jnp.where(kcol == k, sel, acc)
        return m, sel, acc

    v0 = jnp.full((NPOINT, 1), -_FBIG, jnp.float32)
    i0 = jnp.full((NPOINT, 1), -1, jnp.int32)
    acc0 = jnp.zeros((NPOINT, NSAMPLE), jnp.int32)
    _, _, acc = jax.lax.fori_loop(0, NSAMPLE, body, (v0, i0, acc0))
    gi_ref[0] = acc
    q_ref[0] = q


def _knn_pallas(xyzf, fps_idx):
    xrow = xyzf[..., 0].reshape(B, 1, N)
    yrow = xyzf[..., 1].reshape(B, 1, N)
    zrow = xyzf[..., 2].reshape(B, 1, N)
    tab = jnp.concatenate([xyzf, jnp.zeros((B, N, 5), jnp.float32)], axis=-1)
    idxcol = fps_idx.reshape(B, NPOINT, 1)
    gi, q = pl.pallas_call(
        _knn_kernel,
        grid=(B,),
        in_specs=[
            pl.BlockSpec((1, 1, N), lambda b: (b, 0, 0)),
            pl.BlockSpec((1, 1, N), lambda b: (b, 0, 0)),
            pl.BlockSpec((1, 1, N), lambda b: (b, 0, 0)),
            pl.BlockSpec((1, N, 8), lambda b: (b, 0, 0)),
            pl.BlockSpec((1, NPOINT, 1), lambda b: (b, 0, 0)),
        ],
        out_specs=[
            pl.BlockSpec((1, NPOINT, NSAMPLE), lambda b: (b, 0, 0)),
            pl.BlockSpec((1, NPOINT, 8), lambda b: (b, 0, 0)),
        ],
        out_shape=[
            jax.ShapeDtypeStruct((B, NPOINT, NSAMPLE), jnp.int32),
            jax.ShapeDtypeStruct((B, NPOINT, 8), jnp.float32),
        ],
    )(xrow, yrow, zrow, tab, idxcol)
    return gi, q[..., :3]


def _layer_norm(x, g, b):
    m = jnp.mean(x, -1, keepdims=True)
    v = jnp.var(x, -1, keepdims=True)
    return (x - m) / jnp.sqrt(v + EPS) * g + b


def _mha(x, p):
    S, T, D = x.shape
    hd = D // NHEAD
    qkv = jnp.einsum('std,ed->ste', x, p['w_qkv']) + p['b_qkv']
    q, k, v = jnp.split(qkv, 3, axis=-1)

    def heads(a):
        return a.reshape(S, T, NHEAD, hd).transpose(1, 2, 0, 3)

    q, k, v = heads(q), heads(k), heads(v)
    att = jax.nn.softmax(jnp.einsum('thsd,thud->thsu', q, k) / jnp.sqrt(float(hd)), axis=-1)
    o = jnp.einsum('thsu,thud->thsd', att, v).transpose(2, 0, 1, 3).reshape(S, T, D)
    return jnp.einsum('std,ed->ste', o, p['w_o']) + p['b_o']


def _encoder_layer(x, p):
    x = _layer_norm(x + _mha(x, p), p['ln1_g'], p['ln1_b'])
    h = jax.nn.relu(jnp.einsum('std,ed->ste', x, p['w_ff1']) + p['b_ff1'])
    ff = jnp.einsum('ste,de->std', h, p['w_ff2']) + p['b_ff2']
    return _layer_norm(x + ff, p['ln2_g'], p['ln2_b'])


def _conv1x1(x, w, b):
    return jnp.einsum('oc,bc...->bo...', w, x) + b.reshape((1, -1) + (1,) * (x.ndim - 2))


def _pe_net(gx, params):
    h = _conv1x1(gx, params['pe_w1'], params['pe_b1'])
    h = h / jnp.sqrt(1.0 + EPS) * params['pe_bn_g'].reshape(1, -1, 1, 1) + params['pe_bn_b'].reshape(1, -1, 1, 1)
    h = jax.nn.relu(h)
    return _conv1x1(h, params['pe_w2'], params['pe_b2'])


def _lin_interp(x, out_size):
    L = x.shape[-1]
    pos = jnp.arange(out_size) * ((L - 1) / (out_size - 1))
    lo = jnp.floor(pos).astype(jnp.int32)
    hi = jnp.minimum(lo + 1, L - 1)
    w = (pos - lo).astype(x.dtype)
    return x[..., lo] * (1.0 - w) + x[..., hi] * w


def _identity_kernel(x_ref, o_ref):
    o_ref[...] = x_ref[...]


def _pallas_identity(x):
    return pl.pallas_call(
        _identity_kernel,
        out_shape=jax.ShapeDtypeStruct(x.shape, x.dtype),
    )(x)


def kernel(xyz, features, params):
    xyzf = xyz.transpose(0, 2, 1)
    fps_idx = _fps(xyzf, NPOINT)
    group_idx, new_xyz = _knn_pallas(xyzf, fps_idx)

    grouped_xyz = _index_points(xyzf, group_idx).transpose(0, 3, 1, 2)
    grouped_feat = _index_points(features.transpose(0, 2, 1), group_idx).transpose(0, 3, 1, 2)
    inp = grouped_feat + _pe_net(grouped_xyz, params)
    Bb, D, S, K = inp.shape
    x = inp.transpose(0, 2, 1, 3).reshape(Bb * S, D, K).transpose(2, 0, 1)
    for p in params['layers']:
        x = _encoder_layer(x, p)
    tf = x.transpose(1, 2, 0).reshape(Bb, S, D, K).transpose(0, 2, 1, 3)
    pooled = jnp.max(tf, axis=-1)
    out = _conv1x1(pooled, params['fc_w'], params['fc_b'])
    up = _lin_interp(out, N)
    up = _pallas_identity(up)
    return (new_xyz.transpose(0, 2, 1), up)
